# baseline (speedup 1.0000x reference)
import functools

import jax
import jax.numpy as jnp
from jax import lax
from jax.experimental import pallas as pl
from jax.experimental.pallas import tpu as pltpu, tpu_sc as plsc

N = 10000
E = 320000
IN_DIM = 128
HID = 64
OUT = 3

NC = 2
NS = 16
CH = 125
NCHUNK = 80
NBUF = 8
ROWS_PER_TILE = N // NS
TCG = 10

_mesh = plsc.VectorSubcoreMesh(
    core_axis_name="c", subcore_axis_name="s", num_cores=NC, num_subcores=NS
)
_sc_params = pltpu.CompilerParams(use_tc_tiling_on_sc=False)


def _make_deg_kernel():
  @functools.partial(
      pl.kernel,
      out_type=jax.ShapeDtypeStruct((NC, N, 8), jnp.float32),
      mesh=_mesh,
      compiler_params=_sc_params,
      scratch_types=[
          pltpu.VMEM((NCHUNK, CH), jnp.int32),
          pltpu.VMEM((CH, 8), jnp.float32),
          pltpu.SemaphoreType.DMA,
          pltpu.VMEM_SHARED((N, 8), jnp.float32),
      ],
  )
  def deg_kernel(dst_hbm, ones_hbm, zero_hbm, out_hbm, dstb, onesb, sem, accum):
    c = lax.axis_index("c")
    s = lax.axis_index("s")
    row0 = s * ROWS_PER_TILE
    pltpu.sync_copy(
        zero_hbm.at[pl.ds(row0, ROWS_PER_TILE)],
        accum.at[pl.ds(row0, ROWS_PER_TILE)],
    )
    pltpu.sync_copy(dst_hbm.at[c, s], dstb)
    pltpu.sync_copy(ones_hbm, onesb)
    plsc.subcore_barrier()

    @pl.loop(0, NCHUNK, step=16)
    def _(j):
      for u in range(16):
        pltpu.async_copy(onesb, accum.at[dstb.at[j + u]], sem, add=True)
      for u in range(16):
        pltpu.make_async_copy(onesb, accum.at[dstb.at[j + u]], sem).wait()

    plsc.subcore_barrier()
    pltpu.sync_copy(
        accum.at[pl.ds(row0, ROWS_PER_TILE)],
        out_hbm.at[c, pl.ds(row0, ROWS_PER_TILE)],
    )

  return deg_kernel


def _make_prop_kernel(p_passes, w):
  @functools.partial(
      pl.kernel,
      out_type=jax.ShapeDtypeStruct((NC, p_passes, N, w), jnp.float32),
      mesh=_mesh,
      compiler_params=_sc_params,
      scratch_types=[
          pltpu.VMEM((NCHUNK, CH), jnp.int32),
          pltpu.VMEM((NCHUNK, CH), jnp.int32),
          [pltpu.VMEM((CH, w), jnp.float32)] * NBUF,
          [pltpu.SemaphoreType.DMA] * NBUF,
          [pltpu.SemaphoreType.DMA] * NBUF,
          pltpu.VMEM_SHARED((N, w), jnp.float32),
          pltpu.VMEM_SHARED((N, w), jnp.float32),
      ],
  )
  def prop_kernel(y_hbm, src_hbm, dst_hbm, zero_hbm, out_hbm,
                  srcb, dstb, rows, gsem, ssem, accum, y_sp):
    c = lax.axis_index("c")
    s = lax.axis_index("s")
    row0 = s * ROWS_PER_TILE
    pltpu.sync_copy(src_hbm.at[c, s], srcb)
    pltpu.sync_copy(dst_hbm.at[c, s], dstb)

    def gather(j, b):
      pltpu.async_copy(y_sp.at[srcb.at[j]], rows[b], gsem[b])

    def scatter(j, b):
      pltpu.async_copy(rows[b], accum.at[dstb.at[j]], ssem[b], add=True)

    def gather_wait(j, b):
      pltpu.make_async_copy(y_sp.at[srcb.at[j]], rows[b], gsem[b]).wait()

    def scatter_wait(j, b):
      pltpu.make_async_copy(rows[b], accum.at[dstb.at[j]], ssem[b]).wait()

    for p in range(p_passes):
      pltpu.sync_copy(
          y_hbm.at[p, pl.ds(row0, ROWS_PER_TILE)],
          y_sp.at[pl.ds(row0, ROWS_PER_TILE)],
      )
      pltpu.sync_copy(
          zero_hbm.at[pl.ds(row0, ROWS_PER_TILE)],
          accum.at[pl.ds(row0, ROWS_PER_TILE)],
      )
      plsc.subcore_barrier()

      for b in range(NBUF):
        gather(b, b)

      @pl.loop(0, NCHUNK - NBUF, step=NBUF)
      def _(j):
        for b in range(NBUF):
          gather_wait(j + b, b)
          scatter(j + b, b)
        for b in range(NBUF):
          scatter_wait(j + b, b)
          gather(j + b + NBUF, b)

      last = NCHUNK - NBUF
      for b in range(NBUF):
        gather_wait(last + b, b)
        scatter(last + b, b)
      for b in range(NBUF):
        scatter_wait(last + b, b)

      plsc.subcore_barrier()
      pltpu.sync_copy(
          accum.at[pl.ds(row0, ROWS_PER_TILE)],
          out_hbm.at[c, p, pl.ds(row0, ROWS_PER_TILE)],
      )

  return prop_kernel


_deg_kernel = _make_deg_kernel()
_prop64 = _make_prop_kernel(2, HID // 2)
_prop8 = _make_prop_kernel(1, 8)


def _tc_matmul(x, w):
  def body(x_ref, w_ref, o_ref):
    o_ref[...] = jnp.dot(
        x_ref[...], w_ref[...], preferred_element_type=jnp.float32
    )

  blk = N // TCG
  return pl.pallas_call(
      body,
      grid=(TCG,),
      in_specs=[
          pl.BlockSpec((blk, x.shape[1]), lambda i: (i, 0)),
          pl.BlockSpec(w.shape, lambda i: (0, 0)),
      ],
      out_specs=pl.BlockSpec((blk, w.shape[1]), lambda i: (i, 0)),
      out_shape=jax.ShapeDtypeStruct((N, w.shape[1]), jnp.float32),
  )(x, w)


def _tc_scale(xw, degp):
  blk = N // TCG

  def body(xw_ref, degp_ref, y_ref, dinv_ref):
    deg = degp_ref[0, :, 0:1] + degp_ref[1, :, 0:1] + 1.0
    dinv = lax.rsqrt(deg)
    y = xw_ref[...] * dinv
    y_ref[0] = y[:, : HID // 2]
    y_ref[1] = y[:, HID // 2 :]
    dinv_ref[...] = dinv

  return pl.pallas_call(
      body,
      grid=(TCG,),
      in_specs=[
          pl.BlockSpec((blk, HID), lambda i: (i, 0)),
          pl.BlockSpec((NC, blk, 8), lambda i: (0, i, 0)),
      ],
      out_specs=(
          pl.BlockSpec((2, blk, HID // 2), lambda i: (0, i, 0)),
          pl.BlockSpec((blk, 1), lambda i: (i, 0)),
      ),
      out_shape=(
          jax.ShapeDtypeStruct((2, N, HID // 2), jnp.float32),
          jax.ShapeDtypeStruct((N, 1), jnp.float32),
      ),
  )(xw, degp)


def _tc_mid(s, y, dinv, b, w, p_out, w_out):
  def body(s_ref, y_ref, dinv_ref, b_ref, w_ref, yo_ref):
    conv = jnp.concatenate(
        [
            s_ref[0, 0] + s_ref[1, 0] + y_ref[0],
            s_ref[0, 1] + s_ref[1, 1] + y_ref[1],
        ],
        axis=1,
    )
    h = jnp.maximum(dinv_ref[...] * conv + b_ref[...], 0.0)
    yo = (
        jnp.dot(h, w_ref[...], preferred_element_type=jnp.float32)
        * dinv_ref[...]
    )
    if p_out == 1:
      yo_ref[0] = yo
    else:
      yo_ref[0] = yo[:, :w_out]
      yo_ref[1] = yo[:, w_out:]

  blk = N // TCG
  return pl.pallas_call(
      body,
      grid=(TCG,),
      in_specs=[
          pl.BlockSpec((NC, 2, blk, HID // 2), lambda i: (0, 0, i, 0)),
          pl.BlockSpec((2, blk, HID // 2), lambda i: (0, i, 0)),
          pl.BlockSpec((blk, 1), lambda i: (i, 0)),
          pl.BlockSpec((1, HID), lambda i: (0, 0)),
          pl.BlockSpec(w.shape, lambda i: (0, 0)),
      ],
      out_specs=pl.BlockSpec((p_out, blk, w_out), lambda i: (0, i, 0)),
      out_shape=jax.ShapeDtypeStruct((p_out, N, w_out), jnp.float32),
  )(s, y, dinv, b, w)


def _tc_last(s, y, dinv, b):
  def body(s_ref, y_ref, dinv_ref, b_ref, o_ref):
    o_ref[...] = (
        dinv_ref[...] * (s_ref[0, 0] + s_ref[1, 0] + y_ref[0]) + b_ref[...]
    )

  blk = N // TCG
  return pl.pallas_call(
      body,
      grid=(TCG,),
      in_specs=[
          pl.BlockSpec((NC, 1, blk, 8), lambda i: (0, 0, i, 0)),
          pl.BlockSpec((1, blk, 8), lambda i: (0, i, 0)),
          pl.BlockSpec((blk, 1), lambda i: (i, 0)),
          pl.BlockSpec((1, 8), lambda i: (0, 0)),
      ],
      out_specs=pl.BlockSpec((blk, 8), lambda i: (i, 0)),
      out_shape=jax.ShapeDtypeStruct((N, 8), jnp.float32),
  )(s, y, dinv, b)


@jax.jit
def kernel(x, edge_idx, W1, b1, W2, b2, W3, b3):
  src4 = edge_idx[0].astype(jnp.int32).reshape(NC, NS, NCHUNK, CH)
  dst4 = edge_idx[1].astype(jnp.int32).reshape(NC, NS, NCHUNK, CH)

  w3p = jnp.zeros((HID, 8), jnp.float32).at[:, :OUT].set(W3)
  b3p = jnp.zeros((1, 8), jnp.float32).at[0, :OUT].set(b3)
  ones8 = jnp.ones((CH, 8), jnp.float32)
  z32 = jnp.zeros((N, HID // 2), jnp.float32)
  z8 = jnp.zeros((N, 8), jnp.float32)

  degp = _deg_kernel(dst4, ones8, z8)
  xw1 = _tc_matmul(x, W1)
  y1, dinv = _tc_scale(xw1, degp)
  s1 = _prop64(y1, src4, dst4, z32)
  y2 = _tc_mid(s1, y1, dinv, b1.reshape(1, HID), W2, 2, HID // 2)
  s2 = _prop64(y2, src4, dst4, z32)
  y3 = _tc_mid(s2, y2, dinv, b2.reshape(1, HID), w3p, 1, 8)
  s3 = _prop8(y3, src4, dst4, z8)
  outp = _tc_last(s3, y3, dinv, b3p)
  return outp[:, :OUT]


# speedup vs baseline: 1.1671x; 1.1671x over previous
import functools

import jax
import jax.numpy as jnp
from jax import lax
from jax.experimental import pallas as pl
from jax.experimental.pallas import tpu as pltpu, tpu_sc as plsc

N = 10000
E = 320000
IN_DIM = 128
HID = 64
OUT = 3

NC = 2
NS = 16
CH = 128
NCHUNK = 80
NBUF = 8
EPT = NCHUNK * CH
R = 10112
RP = R // 4
ROWS_PER_TILE = R // NS

_mesh = plsc.VectorSubcoreMesh(
    core_axis_name="c", subcore_axis_name="s", num_cores=NC, num_subcores=NS
)
_sc_params = pltpu.CompilerParams(use_tc_tiling_on_sc=False)


def _make_deg_kernel():
  @functools.partial(
      pl.kernel,
      out_type=(
          jax.ShapeDtypeStruct((NC, R, 8), jnp.float32),
          jax.ShapeDtypeStruct((NC, R, 32), jnp.float32),
      ),
      mesh=_mesh,
      compiler_params=_sc_params,
      scratch_types=[
          pltpu.VMEM((NCHUNK, CH), jnp.int32),
          pltpu.VMEM((CH, 8), jnp.float32),
          pltpu.VMEM((CH, 32), jnp.float32),
          pltpu.SemaphoreType.DMA,
          pltpu.SemaphoreType.DMA,
          pltpu.VMEM_SHARED((R, 8), jnp.float32),
          pltpu.VMEM_SHARED((R, 32), jnp.float32),
      ],
  )
  def deg_kernel(dst_hbm, ones8_hbm, ones32_hbm, z8_hbm, z32_hbm,
                 out8_hbm, out32_hbm, dstb, ones8, ones32, sem8, sem32,
                 acc8, acc32):
    c = lax.axis_index("c")
    s = lax.axis_index("s")
    row0 = s * ROWS_PER_TILE
    pltpu.sync_copy(
        z8_hbm.at[pl.ds(row0, ROWS_PER_TILE)],
        acc8.at[pl.ds(row0, ROWS_PER_TILE)],
    )
    pltpu.sync_copy(
        z32_hbm.at[pl.ds(row0, ROWS_PER_TILE)],
        acc32.at[pl.ds(row0, ROWS_PER_TILE)],
    )
    pltpu.sync_copy(dst_hbm.at[c, s], dstb)
    pltpu.sync_copy(ones8_hbm, ones8)
    pltpu.sync_copy(ones32_hbm, ones32)
    plsc.subcore_barrier()

    @pl.loop(0, NCHUNK, step=8)
    def _(j):
      for u in range(8):
        pltpu.async_copy(ones8, acc8.at[dstb.at[j + u]], sem8, add=True)
        pltpu.async_copy(ones32, acc32.at[dstb.at[j + u]], sem32, add=True)
      for u in range(8):
        pltpu.make_async_copy(ones8, acc8.at[dstb.at[j + u]], sem8).wait()
        pltpu.make_async_copy(ones32, acc32.at[dstb.at[j + u]], sem32).wait()

    plsc.subcore_barrier()
    pltpu.sync_copy(
        acc8.at[pl.ds(row0, ROWS_PER_TILE)],
        out8_hbm.at[c, pl.ds(row0, ROWS_PER_TILE)],
    )
    pltpu.sync_copy(
        acc32.at[pl.ds(row0, ROWS_PER_TILE)],
        out32_hbm.at[c, pl.ds(row0, ROWS_PER_TILE)],
    )

  return deg_kernel


def _make_prop_kernel(p_passes, w):
  @functools.partial(
      pl.kernel,
      out_type=jax.ShapeDtypeStruct((NC, p_passes, R, w), jnp.float32),
      mesh=_mesh,
      compiler_params=_sc_params,
      scratch_types=[
          pltpu.VMEM((NCHUNK, CH), jnp.int32),
          pltpu.VMEM((NCHUNK, CH), jnp.int32),
          [pltpu.VMEM((CH, w), jnp.float32)] * NBUF,
          [pltpu.SemaphoreType.DMA] * NBUF,
          [pltpu.SemaphoreType.DMA] * NBUF,
          pltpu.VMEM_SHARED((R, w), jnp.float32),
          pltpu.VMEM_SHARED((R, w), jnp.float32),
      ],
  )
  def prop_kernel(y_hbm, src_hbm, dst_hbm, zero_hbm, out_hbm,
                  srcb, dstb, rows, gsem, ssem, accum, y_sp):
    c = lax.axis_index("c")
    s = lax.axis_index("s")
    row0 = s * ROWS_PER_TILE
    pltpu.sync_copy(src_hbm.at[c, s], srcb)
    pltpu.sync_copy(dst_hbm.at[c, s], dstb)

    def gather(j, b):
      pltpu.async_copy(y_sp.at[srcb.at[j]], rows[b], gsem[b])

    def scatter(j, b):
      pltpu.async_copy(rows[b], accum.at[dstb.at[j]], ssem[b], add=True)

    def gather_wait(j, b):
      pltpu.make_async_copy(y_sp.at[srcb.at[j]], rows[b], gsem[b]).wait()

    def scatter_wait(j, b):
      pltpu.make_async_copy(rows[b], accum.at[dstb.at[j]], ssem[b]).wait()

    for p in range(p_passes):
      pltpu.sync_copy(
          y_hbm.at[p, pl.ds(row0, ROWS_PER_TILE)],
          y_sp.at[pl.ds(row0, ROWS_PER_TILE)],
      )
      pltpu.sync_copy(
          zero_hbm.at[pl.ds(row0, ROWS_PER_TILE)],
          accum.at[pl.ds(row0, ROWS_PER_TILE)],
      )
      plsc.subcore_barrier()

      for b in range(NBUF):
        gather(b, b)

      @pl.loop(0, NCHUNK - NBUF, step=NBUF)
      def _(j):
        for b in range(NBUF):
          gather_wait(j + b, b)
          scatter(j + b, b)
        for b in range(NBUF):
          scatter_wait(j + b, b)
          gather(j + b + NBUF, b)

      last = NCHUNK - NBUF
      for b in range(NBUF):
        gather_wait(last + b, b)
        scatter(last + b, b)
      for b in range(NBUF):
        scatter_wait(last + b, b)

      plsc.subcore_barrier()
      pltpu.sync_copy(
          accum.at[pl.ds(row0, ROWS_PER_TILE)],
          out_hbm.at[c, p, pl.ds(row0, ROWS_PER_TILE)],
      )

  return prop_kernel


_deg_kernel = _make_deg_kernel()
_prop64 = _make_prop_kernel(2, HID // 2)
_prop8 = _make_prop_kernel(1, 8)


def _tc_matmul(x, w):
  def body(x_ref, w_ref, o_ref):
    o_ref[...] = jnp.dot(
        x_ref[...], w_ref[...], preferred_element_type=jnp.float32
    )

  return pl.pallas_call(
      body,
      out_shape=jax.ShapeDtypeStruct((x.shape[0], w.shape[1]), jnp.float32),
  )(x, w)


def _tc_scale(xw, degp8, degp32pk):
  def body(xw_ref, degp8_ref, degp32_ref, y_ref, dinv_ref, dinv128_ref):
    deg = degp8_ref[0, :, 0:1] + degp8_ref[1, :, 0:1] + 1.0
    dinv = lax.rsqrt(deg)
    y = xw_ref[...] * dinv
    y_ref[0] = y[:, : HID // 2]
    y_ref[1] = y[:, HID // 2 :]
    dinv_ref[...] = dinv
    dinv128_ref[...] = lax.rsqrt(degp32_ref[0] + degp32_ref[1] + 1.0)

  return pl.pallas_call(
      body,
      out_shape=(
          jax.ShapeDtypeStruct((2, R, HID // 2), jnp.float32),
          jax.ShapeDtypeStruct((R, 1), jnp.float32),
          jax.ShapeDtypeStruct((RP, 128), jnp.float32),
      ),
  )(xw, degp8, degp32pk)


def _tc_mid_packed(s, y, dinv128, bpk, wbd, q_out, w_out):
  def body(s_ref, y_ref, dinv128_ref, b_ref, w_ref, o_ref):
    dinv = dinv128_ref[...]
    g = []
    for p in range(2):
      u = s_ref[0, p] + s_ref[1, p] + y_ref[p]
      h = jnp.maximum(dinv * u + b_ref[p, 0:1, :], 0.0)
      g.append(dinv * h)
    for q in range(q_out):
      acc = jnp.dot(g[0], w_ref[0, q], preferred_element_type=jnp.float32)
      acc = acc + jnp.dot(g[1], w_ref[1, q],
                          preferred_element_type=jnp.float32)
      o_ref[q] = acc

  return pl.pallas_call(
      body,
      out_shape=jax.ShapeDtypeStruct((q_out, RP, w_out), jnp.float32),
  )(s, y, dinv128, bpk, wbd)


def _tc_last(s, y, dinv, b):
  def body(s_ref, y_ref, dinv_ref, b_ref, o_ref):
    o_ref[...] = (
        dinv_ref[...] * (s_ref[0, 0] + s_ref[1, 0] + y_ref[0]) + b_ref[...]
    )

  return pl.pallas_call(
      body,
      out_shape=jax.ShapeDtypeStruct((R, 8), jnp.float32),
  )(s, y, dinv, b)


def _blockdiag4(wblk):
  return jnp.kron(jnp.eye(4, dtype=jnp.float32), wblk)


@jax.jit
def kernel(x, edge_idx, W1, b1, W2, b2, W3, b3):
  HW = HID // 2
  e2 = edge_idx.astype(jnp.int32).reshape(2, NC * NS, N)
  pad_src = jnp.zeros((NC * NS, EPT - N), jnp.int32)
  pad_dst = jnp.broadcast_to(
      N + (jnp.arange(EPT - N, dtype=jnp.int32) % (R - N)),
      (NC * NS, EPT - N),
  )
  src4 = jnp.concatenate([e2[0], pad_src], axis=1).reshape(NC, NS, NCHUNK, CH)
  dst4 = jnp.concatenate([e2[1], pad_dst], axis=1).reshape(NC, NS, NCHUNK, CH)

  xp = jnp.zeros((R, IN_DIM), jnp.float32).at[:N].set(x)
  w2bd = jnp.stack([
      jnp.stack([_blockdiag4(W2[32 * p : 32 * p + 32, 32 * q : 32 * q + 32])
                 for q in range(2)])
      for p in range(2)
  ])
  w3p = jnp.zeros((HID, 8), jnp.float32).at[:, :OUT].set(W3)
  w3bd = jnp.stack([
      jnp.stack([_blockdiag4(w3p[32 * p : 32 * p + 32, :])])
      for p in range(2)
  ])
  b1pk = jnp.stack([jnp.tile(b1[32 * p : 32 * p + 32], 4).reshape(1, 128)
                    for p in range(2)])
  b2pk = jnp.stack([jnp.tile(b2[32 * p : 32 * p + 32], 4).reshape(1, 128)
                    for p in range(2)])
  b3p = jnp.zeros((1, 8), jnp.float32).at[0, :OUT].set(b3)
  ones8 = jnp.ones((CH, 8), jnp.float32)
  ones32 = jnp.ones((CH, 32), jnp.float32)
  z8 = jnp.zeros((R, 8), jnp.float32)
  z32 = jnp.zeros((R, HW), jnp.float32)

  degp8, degp32 = _deg_kernel(dst4, ones8, ones32, z8, z32)
  xw1 = _tc_matmul(xp, W1)
  y1, dinv, dinv128 = _tc_scale(xw1, degp8, degp32.reshape(NC, RP, 128))

  s1 = _prop64(y1, src4, dst4, z32)
  y2pk = _tc_mid_packed(
      s1.reshape(NC, 2, RP, 128), y1.reshape(2, RP, 128),
      dinv128, b1pk, w2bd, 2, 128,
  )
  s2 = _prop64(y2pk.reshape(2, R, HW), src4, dst4, z32)
  y3pk = _tc_mid_packed(
      s2.reshape(NC, 2, RP, 128), y2pk,
      dinv128, b2pk, w3bd, 1, 32,
  )
  y3 = y3pk.reshape(1, R, 8)
  s3 = _prop8(y3, src4, dst4, z8)
  outp = _tc_last(s3, y3, dinv, b3p)
  return outp[:N, :OUT]


# speedup vs baseline: 1.2645x; 1.0835x over previous
import functools

import jax
import jax.numpy as jnp
from jax import lax
from jax.experimental import pallas as pl
from jax.experimental.pallas import tpu as pltpu, tpu_sc as plsc

N = 10000
E = 320000
IN_DIM = 128
HID = 64
OUT = 3

NC = 2
NS = 16
CH = 128
NCHUNK = 80
NBUF = 8
EPT = NCHUNK * CH
R = 10112
RP = R // 4
RPL = R // 16
ROWS_PER_TILE = R // NS

_mesh = plsc.VectorSubcoreMesh(
    core_axis_name="c", subcore_axis_name="s", num_cores=NC, num_subcores=NS
)
_sc_params = pltpu.CompilerParams(use_tc_tiling_on_sc=False)


def _make_deg_kernel():
  @functools.partial(
      pl.kernel,
      out_type=(
          jax.ShapeDtypeStruct((NC, R, 8), jnp.float32),
          jax.ShapeDtypeStruct((NC, R, 32), jnp.float32),
      ),
      mesh=_mesh,
      compiler_params=_sc_params,
      scratch_types=[
          pltpu.VMEM((NCHUNK, CH), jnp.int32),
          pltpu.VMEM((CH, 8), jnp.float32),
          pltpu.VMEM((CH, 32), jnp.float32),
          pltpu.SemaphoreType.DMA,
          pltpu.SemaphoreType.DMA,
          pltpu.VMEM_SHARED((R, 8), jnp.float32),
          pltpu.VMEM_SHARED((R, 32), jnp.float32),
      ],
  )
  def deg_kernel(dst_hbm, ones8_hbm, ones32_hbm, z8_hbm, z32_hbm,
                 out8_hbm, out32_hbm, dstb, ones8, ones32, sem8, sem32,
                 acc8, acc32):
    c = lax.axis_index("c")
    s = lax.axis_index("s")
    row0 = s * ROWS_PER_TILE
    pltpu.sync_copy(
        z8_hbm.at[pl.ds(row0, ROWS_PER_TILE)],
        acc8.at[pl.ds(row0, ROWS_PER_TILE)],
    )
    pltpu.sync_copy(
        z32_hbm.at[pl.ds(row0, ROWS_PER_TILE)],
        acc32.at[pl.ds(row0, ROWS_PER_TILE)],
    )
    pltpu.sync_copy(dst_hbm.at[c, s], dstb)
    pltpu.sync_copy(ones8_hbm, ones8)
    pltpu.sync_copy(ones32_hbm, ones32)
    plsc.subcore_barrier()

    @pl.loop(0, NCHUNK, step=8)
    def _(j):
      for u in range(8):
        pltpu.async_copy(ones8, acc8.at[dstb.at[j + u]], sem8, add=True)
        pltpu.async_copy(ones32, acc32.at[dstb.at[j + u]], sem32, add=True)
      for u in range(8):
        pltpu.make_async_copy(ones8, acc8.at[dstb.at[j + u]], sem8).wait()
        pltpu.make_async_copy(ones32, acc32.at[dstb.at[j + u]], sem32).wait()

    plsc.subcore_barrier()
    pltpu.sync_copy(
        acc8.at[pl.ds(row0, ROWS_PER_TILE)],
        out8_hbm.at[c, pl.ds(row0, ROWS_PER_TILE)],
    )
    pltpu.sync_copy(
        acc32.at[pl.ds(row0, ROWS_PER_TILE)],
        out32_hbm.at[c, pl.ds(row0, ROWS_PER_TILE)],
    )

  return deg_kernel


def _make_prop_kernel(p_passes, w):
  @functools.partial(
      pl.kernel,
      out_type=jax.ShapeDtypeStruct((NC, p_passes, R, w), jnp.float32),
      mesh=_mesh,
      compiler_params=_sc_params,
      scratch_types=[
          pltpu.VMEM((NCHUNK, CH), jnp.int32),
          pltpu.VMEM((NCHUNK, CH), jnp.int32),
          [pltpu.VMEM((CH, w), jnp.float32)] * NBUF,
          [pltpu.SemaphoreType.DMA] * NBUF,
          [pltpu.SemaphoreType.DMA] * NBUF,
          pltpu.VMEM_SHARED((R, w), jnp.float32),
          pltpu.VMEM_SHARED((R, w), jnp.float32),
      ],
  )
  def prop_kernel(y_hbm, src_hbm, dst_hbm, zero_hbm, out_hbm,
                  srcb, dstb, rows, gsem, ssem, accum, y_sp):
    c = lax.axis_index("c")
    s = lax.axis_index("s")
    row0 = s * ROWS_PER_TILE
    pltpu.sync_copy(src_hbm.at[c, s], srcb)
    pltpu.sync_copy(dst_hbm.at[c, s], dstb)

    def gather(j, b):
      pltpu.async_copy(y_sp.at[srcb.at[j]], rows[b], gsem[b])

    def scatter(j, b):
      pltpu.async_copy(rows[b], accum.at[dstb.at[j]], ssem[b], add=True)

    def gather_wait(j, b):
      pltpu.make_async_copy(y_sp.at[srcb.at[j]], rows[b], gsem[b]).wait()

    def scatter_wait(j, b):
      pltpu.make_async_copy(rows[b], accum.at[dstb.at[j]], ssem[b]).wait()

    for p in range(p_passes):
      pltpu.sync_copy(
          y_hbm.at[p, pl.ds(row0, ROWS_PER_TILE)],
          y_sp.at[pl.ds(row0, ROWS_PER_TILE)],
      )
      pltpu.sync_copy(
          zero_hbm.at[pl.ds(row0, ROWS_PER_TILE)],
          accum.at[pl.ds(row0, ROWS_PER_TILE)],
      )
      plsc.subcore_barrier()

      for b in range(NBUF):
        gather(b, b)

      @pl.loop(0, NCHUNK - NBUF, step=NBUF)
      def _(j):
        for b in range(NBUF):
          gather_wait(j + b, b)
          scatter(j + b, b)
        for b in range(NBUF):
          scatter_wait(j + b, b)
          gather(j + b + NBUF, b)

      last = NCHUNK - NBUF
      for b in range(NBUF):
        gather_wait(last + b, b)
        scatter(last + b, b)
      for b in range(NBUF):
        scatter_wait(last + b, b)

      plsc.subcore_barrier()
      pltpu.sync_copy(
          accum.at[pl.ds(row0, ROWS_PER_TILE)],
          out_hbm.at[c, p, pl.ds(row0, ROWS_PER_TILE)],
      )

  return prop_kernel


_deg_kernel = _make_deg_kernel()
_prop64 = _make_prop_kernel(2, HID // 2)
_prop8 = _make_prop_kernel(1, 8)


def _tc_matmul(x, w):
  def body(x_ref, w_ref, o_ref):
    o_ref[...] = jnp.dot(
        x_ref[...], w_ref[...], preferred_element_type=jnp.float32
    )

  return pl.pallas_call(
      body,
      out_shape=jax.ShapeDtypeStruct((x.shape[0], w.shape[1]), jnp.float32),
  )(x, w)


def _tc_scale(xwpk, degp8pk, degp32pk):
  def body(xw_ref, degp8_ref, degp32_ref, y_ref, dinv128_ref, dinv8_ref):
    dinv128 = lax.rsqrt(degp32_ref[0] + degp32_ref[1] + 1.0)
    y_ref[0] = xw_ref[0] * dinv128
    y_ref[1] = xw_ref[1] * dinv128
    dinv128_ref[...] = dinv128
    dinv8_ref[...] = lax.rsqrt(degp8_ref[0] + degp8_ref[1] + 1.0)

  return pl.pallas_call(
      body,
      out_shape=(
          jax.ShapeDtypeStruct((2, RP, 128), jnp.float32),
          jax.ShapeDtypeStruct((RP, 128), jnp.float32),
          jax.ShapeDtypeStruct((RPL, 128), jnp.float32),
      ),
  )(xwpk, degp8pk, degp32pk)


def _tc_mid_packed(s, y, dinv128, bpk, wbd, q_out, w_out):
  def body(s_ref, y_ref, dinv128_ref, b_ref, w_ref, o_ref):
    dinv = dinv128_ref[...]
    g = []
    for p in range(2):
      u = s_ref[0, p] + s_ref[1, p] + y_ref[p]
      h = jnp.maximum(dinv * u + b_ref[p, 0:1, :], 0.0)
      g.append(dinv * h)
    for q in range(q_out):
      acc = jnp.dot(g[0], w_ref[0, q], preferred_element_type=jnp.float32)
      acc = acc + jnp.dot(g[1], w_ref[1, q],
                          preferred_element_type=jnp.float32)
      o_ref[q] = acc

  return pl.pallas_call(
      body,
      out_shape=jax.ShapeDtypeStruct((q_out, RP, w_out), jnp.float32),
  )(s, y, dinv128, bpk, wbd)


def _tc_last(s, y, dinv8, b):
  def body(s_ref, y_ref, dinv8_ref, b_ref, o_ref):
    o_ref[...] = (
        dinv8_ref[...] * (s_ref[0] + s_ref[1] + y_ref[...]) + b_ref[...]
    )

  return pl.pallas_call(
      body,
      out_shape=jax.ShapeDtypeStruct((RPL, 128), jnp.float32),
  )(s, y, dinv8, b)


def _blockdiag4(wblk):
  return jnp.kron(jnp.eye(4, dtype=jnp.float32), wblk)


@jax.jit
def kernel(x, edge_idx, W1, b1, W2, b2, W3, b3):
  HW = HID // 2
  e2 = edge_idx.astype(jnp.int32).reshape(2, NC * NS, N)
  pad_src = jnp.zeros((NC * NS, EPT - N), jnp.int32)
  pad_dst = jnp.broadcast_to(
      N + (jnp.arange(EPT - N, dtype=jnp.int32) % (R - N)),
      (NC * NS, EPT - N),
  )
  src4 = jnp.concatenate([e2[0], pad_src], axis=1).reshape(NC, NS, NCHUNK, CH)
  dst4 = jnp.concatenate([e2[1], pad_dst], axis=1).reshape(NC, NS, NCHUNK, CH)

  xp = jnp.zeros((R, IN_DIM), jnp.float32).at[:N].set(x)
  w2bd = jnp.stack([
      jnp.stack([_blockdiag4(W2[32 * p : 32 * p + 32, 32 * q : 32 * q + 32])
                 for q in range(2)])
      for p in range(2)
  ])
  w3p = jnp.zeros((HID, 8), jnp.float32).at[:, :OUT].set(W3)
  w3bd = jnp.stack([
      jnp.stack([_blockdiag4(w3p[32 * p : 32 * p + 32, :])])
      for p in range(2)
  ])
  b1pk = jnp.stack([jnp.tile(b1[32 * p : 32 * p + 32], 4).reshape(1, 128)
                    for p in range(2)])
  b2pk = jnp.stack([jnp.tile(b2[32 * p : 32 * p + 32], 4).reshape(1, 128)
                    for p in range(2)])
  b3p = jnp.zeros((1, 8), jnp.float32).at[0, :OUT].set(b3)
  b3pk = jnp.tile(b3p, (1, 16))
  ones8 = jnp.ones((CH, 8), jnp.float32)
  ones32 = jnp.ones((CH, 32), jnp.float32)
  z8 = jnp.zeros((R, 8), jnp.float32)
  z32 = jnp.zeros((R, HW), jnp.float32)

  degp8, degp32 = _deg_kernel(dst4, ones8, ones32, z8, z32)
  xw1 = _tc_matmul(xp, W1)
  xw1pk = jnp.stack([xw1[:, :HW], xw1[:, HW:]]).reshape(2, RP, 128)
  y1, dinv128, dinv8 = _tc_scale(
      xw1pk, degp8.reshape(NC, RPL, 128), degp32.reshape(NC, RP, 128)
  )

  s1 = _prop64(y1.reshape(2, R, HW), src4, dst4, z32)
  y2pk = _tc_mid_packed(
      s1.reshape(NC, 2, RP, 128), y1,
      dinv128, b1pk, w2bd, 2, 128,
  )
  s2 = _prop64(y2pk.reshape(2, R, HW), src4, dst4, z32)
  y3pk = _tc_mid_packed(
      s2.reshape(NC, 2, RP, 128), y2pk,
      dinv128, b2pk, w3bd, 1, 32,
  )
  y3g = y3pk.reshape(RPL, 128)
  s3 = _prop8(y3g.reshape(1, R, 8), src4, dst4, z8)
  outp = _tc_last(s3.reshape(NC, RPL, 128), y3g, dinv8, b3pk)
  return outp.reshape(R, 8)[:N, :OUT]


# speedup vs baseline: 1.3199x; 1.0438x over previous
import functools

import jax
import jax.numpy as jnp
from jax import lax
from jax.experimental import pallas as pl
from jax.experimental.pallas import tpu as pltpu, tpu_sc as plsc

N = 10000
E = 320000
IN_DIM = 128
HID = 64
OUT = 3

NC = 2
NS = 16
CH = 128
NCHUNK = 80
NBUF = 8
EPT = NCHUNK * CH
R = 10112
RP = R // 4
RPL = R // 16
ROWS_PER_TILE = R // NS

_mesh = plsc.VectorSubcoreMesh(
    core_axis_name="c", subcore_axis_name="s", num_cores=NC, num_subcores=NS
)
_sc_params = pltpu.CompilerParams(use_tc_tiling_on_sc=False)


def _make_deg_kernel():
  @functools.partial(
      pl.kernel,
      out_type=(
          jax.ShapeDtypeStruct((NC, R, 8), jnp.float32),
          jax.ShapeDtypeStruct((NC, R, 32), jnp.float32),
      ),
      mesh=_mesh,
      compiler_params=_sc_params,
      scratch_types=[
          pltpu.VMEM((NCHUNK, CH), jnp.int32),
          pltpu.VMEM((CH, 8), jnp.float32),
          pltpu.VMEM((CH, 32), jnp.float32),
          pltpu.SemaphoreType.DMA,
          pltpu.SemaphoreType.DMA,
          pltpu.VMEM_SHARED((R, 8), jnp.float32),
          pltpu.VMEM_SHARED((R, 32), jnp.float32),
      ],
  )
  def deg_kernel(dst_hbm, ones8_hbm, ones32_hbm, z8_hbm, z32_hbm,
                 out8_hbm, out32_hbm, dstb, ones8, ones32, sem8, sem32,
                 acc8, acc32):
    c = lax.axis_index("c")
    s = lax.axis_index("s")
    row0 = s * ROWS_PER_TILE
    pltpu.sync_copy(
        z8_hbm.at[pl.ds(row0, ROWS_PER_TILE)],
        acc8.at[pl.ds(row0, ROWS_PER_TILE)],
    )
    pltpu.sync_copy(
        z32_hbm.at[pl.ds(row0, ROWS_PER_TILE)],
        acc32.at[pl.ds(row0, ROWS_PER_TILE)],
    )
    pltpu.sync_copy(dst_hbm.at[c, s], dstb)
    pltpu.sync_copy(ones8_hbm, ones8)
    pltpu.sync_copy(ones32_hbm, ones32)
    plsc.subcore_barrier()

    @pl.loop(0, NCHUNK, step=8)
    def _(j):
      for u in range(8):
        pltpu.async_copy(ones8, acc8.at[dstb.at[j + u]], sem8, add=True)
        pltpu.async_copy(ones32, acc32.at[dstb.at[j + u]], sem32, add=True)
      for u in range(8):
        pltpu.make_async_copy(ones8, acc8.at[dstb.at[j + u]], sem8).wait()
        pltpu.make_async_copy(ones32, acc32.at[dstb.at[j + u]], sem32).wait()

    plsc.subcore_barrier()
    pltpu.sync_copy(
        acc8.at[pl.ds(row0, ROWS_PER_TILE)],
        out8_hbm.at[c, pl.ds(row0, ROWS_PER_TILE)],
    )
    pltpu.sync_copy(
        acc32.at[pl.ds(row0, ROWS_PER_TILE)],
        out32_hbm.at[c, pl.ds(row0, ROWS_PER_TILE)],
    )

  return deg_kernel


def _make_prop_kernel(p_passes, w):
  @functools.partial(
      pl.kernel,
      out_type=jax.ShapeDtypeStruct((NC, p_passes, R, w), jnp.float32),
      mesh=_mesh,
      compiler_params=_sc_params,
      scratch_types=[
          pltpu.VMEM((NCHUNK, CH), jnp.int32),
          pltpu.VMEM((NCHUNK, CH), jnp.int32),
          [pltpu.VMEM((CH, w), jnp.float32)] * NBUF,
          [pltpu.SemaphoreType.DMA] * NBUF,
          [pltpu.SemaphoreType.DMA] * NBUF,
          pltpu.VMEM_SHARED((R, w), jnp.float32),
          pltpu.VMEM_SHARED((R, w), jnp.float32),
      ],
  )
  def prop_kernel(y_hbm, src_hbm, dst_hbm, zero_hbm, out_hbm,
                  srcb, dstb, rows, gsem, ssem, accum, y_sp):
    c = lax.axis_index("c")
    s = lax.axis_index("s")
    row0 = s * ROWS_PER_TILE
    pltpu.sync_copy(src_hbm.at[c, s], srcb)
    pltpu.sync_copy(dst_hbm.at[c, s], dstb)

    def gather(j, b):
      pltpu.async_copy(y_sp.at[srcb.at[j]], rows[b], gsem[b])

    def scatter(j, b):
      pltpu.async_copy(rows[b], accum.at[dstb.at[j]], ssem[b], add=True)

    def gather_wait(j, b):
      pltpu.make_async_copy(y_sp.at[srcb.at[j]], rows[b], gsem[b]).wait()

    def scatter_wait(j, b):
      pltpu.make_async_copy(rows[b], accum.at[dstb.at[j]], ssem[b]).wait()

    for p in range(p_passes):
      pltpu.sync_copy(
          y_hbm.at[p, pl.ds(row0, ROWS_PER_TILE)],
          y_sp.at[pl.ds(row0, ROWS_PER_TILE)],
      )
      pltpu.sync_copy(
          zero_hbm.at[pl.ds(row0, ROWS_PER_TILE)],
          accum.at[pl.ds(row0, ROWS_PER_TILE)],
      )
      plsc.subcore_barrier()

      for b in range(NBUF):
        gather(b, b)

      @pl.loop(0, NCHUNK - NBUF, step=NBUF)
      def _(j):
        for b in range(NBUF):
          gather_wait(j + b, b)
          scatter(j + b, b)
        for b in range(NBUF):
          scatter_wait(j + b, b)
          gather(j + b + NBUF, b)

      last = NCHUNK - NBUF
      for b in range(NBUF):
        gather_wait(last + b, b)
        scatter(last + b, b)
      for b in range(NBUF):
        scatter_wait(last + b, b)

      plsc.subcore_barrier()
      pltpu.sync_copy(
          accum.at[pl.ds(row0, ROWS_PER_TILE)],
          out_hbm.at[c, p, pl.ds(row0, ROWS_PER_TILE)],
      )

  return prop_kernel


_deg_kernel = _make_deg_kernel()
_prop64 = _make_prop_kernel(2, HID // 2)
_prop8 = _make_prop_kernel(1, 8)


def _tc_matmul(x, w):
  def body(x_ref, w_ref, o_ref):
    o_ref[...] = jnp.dot(
        x_ref[...], w_ref[...], preferred_element_type=jnp.float32
    )

  return pl.pallas_call(
      body,
      out_shape=jax.ShapeDtypeStruct((x.shape[0], w.shape[1]), jnp.float32),
  )(x, w)


def _tc_scale(xwpk, degp8pk, degp32pk):
  def body(xw_ref, degp8_ref, degp32_ref, y_ref, dinv128_ref, dinv8_ref):
    dinv128 = lax.rsqrt(degp32_ref[0] + degp32_ref[1] + 1.0)
    y_ref[0] = xw_ref[0] * dinv128
    y_ref[1] = xw_ref[1] * dinv128
    dinv128_ref[...] = dinv128
    dinv8_ref[...] = lax.rsqrt(degp8_ref[0] + degp8_ref[1] + 1.0)

  return pl.pallas_call(
      body,
      out_shape=(
          jax.ShapeDtypeStruct((2, RP, 128), jnp.float32),
          jax.ShapeDtypeStruct((RP, 128), jnp.float32),
          jax.ShapeDtypeStruct((RPL, 128), jnp.float32),
      ),
  )(xwpk, degp8pk, degp32pk)


def _tc_mid_packed(s, y, dinv128, bpk, wbd, q_out, w_out):
  def body(s_ref, y_ref, dinv128_ref, b_ref, w_ref, o_ref):
    dinv = dinv128_ref[...]
    g = []
    for p in range(2):
      u = s_ref[0, p] + s_ref[1, p] + y_ref[p]
      h = jnp.maximum(dinv * u + b_ref[p, 0:1, :], 0.0)
      g.append(dinv * h)
    for q in range(q_out):
      acc = jnp.dot(g[0], w_ref[0, q], preferred_element_type=jnp.float32)
      acc = acc + jnp.dot(g[1], w_ref[1, q],
                          preferred_element_type=jnp.float32)
      o_ref[q] = acc

  return pl.pallas_call(
      body,
      out_shape=jax.ShapeDtypeStruct((q_out, RP, w_out), jnp.float32),
  )(s, y, dinv128, bpk, wbd)


def _tc_last(s, y, dinv8, b):
  def body(s_ref, y_ref, dinv8_ref, b_ref, o_ref):
    o_ref[...] = (
        dinv8_ref[...] * (s_ref[0] + s_ref[1] + y_ref[...]) + b_ref[...]
    )

  return pl.pallas_call(
      body,
      out_shape=jax.ShapeDtypeStruct((RPL, 128), jnp.float32),
  )(s, y, dinv8, b)


def _blockdiag4(wblk):
  return jnp.kron(jnp.eye(4, dtype=jnp.float32), wblk)


@jax.jit
def kernel(x, edge_idx, W1, b1, W2, b2, W3, b3):
  HW = HID // 2
  e2 = edge_idx.astype(jnp.int32).reshape(2, NC * NS, N)
  pad_dst = N + (jnp.arange(EPT - N, dtype=jnp.int32) % (R - N))
  pads = jnp.broadcast_to(
      jnp.stack([jnp.zeros_like(pad_dst), pad_dst])[:, None, :],
      (2, NC * NS, EPT - N),
  )
  e4 = jnp.concatenate([e2, pads], axis=2).reshape(2, NC, NS, NCHUNK, CH)
  src4, dst4 = e4[0], e4[1]

  xp = jnp.zeros((R, IN_DIM), jnp.float32).at[:N].set(x)
  w2bd = jnp.stack([
      jnp.stack([_blockdiag4(W2[32 * p : 32 * p + 32, 32 * q : 32 * q + 32])
                 for q in range(2)])
      for p in range(2)
  ])
  w3p = jnp.zeros((HID, 8), jnp.float32).at[:, :OUT].set(W3)
  w3bd = jnp.stack([
      jnp.stack([_blockdiag4(w3p[32 * p : 32 * p + 32, :])])
      for p in range(2)
  ])
  b1pk = jnp.stack([jnp.tile(b1[32 * p : 32 * p + 32], 4).reshape(1, 128)
                    for p in range(2)])
  b2pk = jnp.stack([jnp.tile(b2[32 * p : 32 * p + 32], 4).reshape(1, 128)
                    for p in range(2)])
  b3p = jnp.zeros((1, 8), jnp.float32).at[0, :OUT].set(b3)
  b3pk = jnp.tile(b3p, (1, 16))
  ones8 = jnp.ones((CH, 8), jnp.float32)
  ones32 = jnp.ones((CH, 32), jnp.float32)
  z8 = jnp.zeros((R, 8), jnp.float32)
  z32 = jnp.zeros((R, HW), jnp.float32)

  degp8, degp32 = _deg_kernel(dst4, ones8, ones32, z8, z32)
  xw1 = _tc_matmul(xp, W1)
  xw1pk = jnp.stack([xw1[:, :HW], xw1[:, HW:]]).reshape(2, RP, 128)
  y1, dinv128, dinv8 = _tc_scale(
      xw1pk, degp8.reshape(NC, RPL, 128), degp32.reshape(NC, RP, 128)
  )

  s1 = _prop64(y1.reshape(2, R, HW), src4, dst4, z32)
  y2pk = _tc_mid_packed(
      s1.reshape(NC, 2, RP, 128), y1,
      dinv128, b1pk, w2bd, 2, 128,
  )
  s2 = _prop64(y2pk.reshape(2, R, HW), src4, dst4, z32)
  y3pk = _tc_mid_packed(
      s2.reshape(NC, 2, RP, 128), y2pk,
      dinv128, b2pk, w3bd, 1, 32,
  )
  y3g = y3pk.reshape(RPL, 128)
  s3 = _prop8(y3g.reshape(1, R, 8), src4, dst4, z8)
  outp = _tc_last(s3.reshape(NC, RPL, 128), y3g, dinv8, b3pk)
  return outp.reshape(R, 8)[:N, :OUT]
